# baseline (device time: 29492 ns/iter reference)
import jax
import jax.numpy as jnp
from jax import lax
from jax.experimental import pallas as pl
from jax.experimental.pallas import tpu as pltpu

N_DEV = 4
B = 2
SQ = 128
SKV = 128
D = 512
H = 8
DH = 64
G = B * H
HGR = (G // 2) * DH


def kernel(x, Wq, Wo, K_ext, V_ext):
    x2d = x.reshape(B * SQ, D)
    kt = K_ext.transpose(0, 2, 3, 1).reshape(G * DH, SKV)
    vt = V_ext.transpose(0, 2, 3, 1).reshape(G * DH, SKV)

    def body(x_ref, wq_ref, wo_ref, k_ref, v_ref, out_ref,
             q2d, qg, sscr, acc,
             kn1, kp1, kd, vn1, vp1, vd, send_sems, recv_sems):
        my = lax.axis_index("i")
        left = lax.rem(my + N_DEV - 1, N_DEV)
        right = lax.rem(my + 1, N_DEV)

        barrier = pltpu.get_barrier_semaphore()
        for nbr in (left, right):
            pl.semaphore_signal(barrier, inc=1, device_id=(nbr,),
                                device_id_type=pl.DeviceIdType.MESH)
        pl.semaphore_wait(barrier, 2)

        def rc(i, src, dst, dev):
            return pltpu.make_async_remote_copy(
                src_ref=src, dst_ref=dst,
                send_sem=send_sems.at[i], recv_sem=recv_sems.at[i],
                device_id=(dev,), device_id_type=pl.DeviceIdType.MESH)

        f = [None] * 8
        f[0] = rc(0, k_ref, kn1, right)
        f[2] = rc(2, k_ref, kp1, left)
        f[1] = rc(1, v_ref, vn1, right)
        f[3] = rc(3, v_ref, vp1, left)
        for i in (0, 2, 1, 3):
            f[i].start()

        q2d[...] = jnp.dot(x_ref[...], wq_ref[...],
                           preferred_element_type=jnp.float32)
        for g in range(G):
            b, hh = divmod(g, H)
            qg[g] = q2d[b * SQ:(b + 1) * SQ, hh * DH:(hh + 1) * DH]

        def qk(slot, kbuf):
            kt3 = jnp.reshape(kbuf, (G, DH, SKV))
            s = lax.dot_general(
                qg[...], kt3, (((2,), (1,)), ((0,), (0,))),
                preferred_element_type=jnp.float32)
            sscr[:, :, slot * SKV:(slot + 1) * SKV] = s * 0.125

        qk(0, k_ref[...])

        f[0].wait_recv()
        f[4] = rc(4, kn1.at[0:HGR], kd.at[0:HGR], right)
        f[4].start()
        qk(1, kn1[...])
        f[2].wait_recv()
        f[6] = rc(6, kp1.at[HGR:2 * HGR], kd.at[HGR:2 * HGR], left)
        f[6].start()
        qk(2, kp1[...])
        f[1].wait_recv()
        f[5] = rc(5, vn1.at[0:HGR], vd.at[0:HGR], right)
        f[5].start()
        f[3].wait_recv()
        f[7] = rc(7, vp1.at[HGR:2 * HGR], vd.at[HGR:2 * HGR], left)
        f[7].start()

        f[4].wait_recv()
        f[6].wait_recv()
        qk(3, kd[...])

        for g in range(G):
            sg = sscr[g]
            m = jnp.max(sg, axis=1, keepdims=True)
            p = jnp.exp(sg - m)
            l = jnp.sum(p, axis=1, keepdims=True)
            sscr[g] = p * (1.0 / l)

        f[5].wait_recv()
        f[7].wait_recv()

        def pv(slot, vbuf):
            vt3 = jnp.reshape(vbuf, (G, DH, SKV))
            return lax.dot_general(
                sscr[:, :, slot * SKV:(slot + 1) * SKV], vt3,
                (((2,), (2,)), ((0,), (0,))),
                preferred_element_type=jnp.float32)

        o = pv(0, v_ref[...])
        o = o + pv(1, vn1[...])
        o = o + pv(2, vp1[...])
        o = o + pv(3, vd[...])
        for g in range(G):
            b, hh = divmod(g, H)
            acc[b * SQ:(b + 1) * SQ, hh * DH:(hh + 1) * DH] = o[g]

        out_ref[...] = jnp.dot(acc[...], wo_ref[...],
                               preferred_element_type=jnp.float32)

        for i in range(8):
            f[i].wait_send()

    out2d = pl.pallas_call(
        body,
        out_shape=jax.ShapeDtypeStruct((B * SQ, D), jnp.float32),
        in_specs=[pl.BlockSpec(memory_space=pltpu.VMEM)] * 5,
        out_specs=pl.BlockSpec(memory_space=pltpu.VMEM),
        scratch_shapes=[
            pltpu.VMEM((B * SQ, D), jnp.float32),
            pltpu.VMEM((G, SQ, DH), jnp.float32),
            pltpu.VMEM((G, SQ, N_DEV * SKV), jnp.float32),
            pltpu.VMEM((B * SQ, D), jnp.float32),
            pltpu.VMEM((G * DH, SKV), jnp.float32),
            pltpu.VMEM((G * DH, SKV), jnp.float32),
            pltpu.VMEM((G * DH, SKV), jnp.float32),
            pltpu.VMEM((G * DH, SKV), jnp.float32),
            pltpu.VMEM((G * DH, SKV), jnp.float32),
            pltpu.VMEM((G * DH, SKV), jnp.float32),
            pltpu.SemaphoreType.DMA((8,)),
            pltpu.SemaphoreType.DMA((8,)),
        ],
        compiler_params=pltpu.CompilerParams(collective_id=0),
    )(x2d, Wq, Wo, kt, vt)
    return out2d.reshape(B, SQ, D)


# device time: 27934 ns/iter; 1.0558x vs baseline; 1.0558x over previous
import jax
import jax.numpy as jnp
from jax import lax
from jax.experimental import pallas as pl
from jax.experimental.pallas import tpu as pltpu

N_DEV = 4
B = 2
SQ = 128
SKV = 128
D = 512
H = 8
DH = 64
G = B * H
HGR = (G // 2) * DH


def kernel(x, Wq, Wo, K_ext, V_ext):
    x2d = x.reshape(B * SQ, D)
    kt = K_ext.transpose(0, 2, 3, 1).reshape(G * DH, SKV)
    vt = V_ext.transpose(0, 2, 3, 1).reshape(G * DH, SKV)

    def body(x_ref, wq_ref, wo_ref, k_ref, v_ref, out_ref,
             q2d, qg, sscr, acc,
             kn1, kp1, kd, vn1, vp1, vd, send_sems, recv_sems):
        my = lax.axis_index("i")
        left = lax.rem(my + N_DEV - 1, N_DEV)
        right = lax.rem(my + 1, N_DEV)

        barrier = pltpu.get_barrier_semaphore()
        for nbr in (left, right):
            pl.semaphore_signal(barrier, inc=1, device_id=(nbr,),
                                device_id_type=pl.DeviceIdType.MESH)
        pl.semaphore_wait(barrier, 2)

        def rc(i, src, dst, dev):
            return pltpu.make_async_remote_copy(
                src_ref=src, dst_ref=dst,
                send_sem=send_sems.at[i], recv_sem=recv_sems.at[i],
                device_id=(dev,), device_id_type=pl.DeviceIdType.MESH)

        f = [None] * 8
        f[0] = rc(0, k_ref, kn1, right)
        f[2] = rc(2, k_ref, kp1, left)
        f[1] = rc(1, v_ref, vn1, right)
        f[3] = rc(3, v_ref, vp1, left)
        for i in (0, 2, 1, 3):
            f[i].start()

        q2d[...] = jnp.dot(x_ref[...], wq_ref[...],
                           preferred_element_type=jnp.float32)
        for g in range(G):
            b, hh = divmod(g, H)
            qg[g] = q2d[b * SQ:(b + 1) * SQ, hh * DH:(hh + 1) * DH]

        def qk(slot, kbuf):
            kt3 = jnp.reshape(kbuf, (G, DH, SKV))
            s = lax.dot_general(
                qg[...], kt3, (((2,), (1,)), ((0,), (0,))),
                preferred_element_type=jnp.float32)
            sscr[:, :, slot * SKV:(slot + 1) * SKV] = s * 0.125

        qk(0, k_ref[...])

        f[0].wait_recv()
        f[4] = rc(4, kn1.at[0:HGR], kd.at[0:HGR], right)
        f[4].start()
        qk(1, kn1[...])
        f[2].wait_recv()
        f[6] = rc(6, kp1.at[HGR:2 * HGR], kd.at[HGR:2 * HGR], left)
        f[6].start()
        qk(2, kp1[...])
        f[1].wait_recv()
        f[5] = rc(5, vn1.at[0:HGR], vd.at[0:HGR], right)
        f[5].start()
        f[3].wait_recv()
        f[7] = rc(7, vp1.at[HGR:2 * HGR], vd.at[HGR:2 * HGR], left)
        f[7].start()

        f[4].wait_recv()
        f[6].wait_recv()
        qk(3, kd[...])

        for g in range(G):
            sg = sscr[g]
            m = jnp.max(sg, axis=1, keepdims=True)
            p = jnp.exp(sg - m)
            l = jnp.sum(p, axis=1, keepdims=True)
            sscr[g] = p * (1.0 / l)

        def pv(slot, vbuf):
            vt3 = jnp.reshape(vbuf, (G, DH, SKV))
            return lax.dot_general(
                sscr[:, :, slot * SKV:(slot + 1) * SKV], vt3,
                (((2,), (2,)), ((0,), (0,))),
                preferred_element_type=jnp.float32)

        o = pv(0, v_ref[...])
        o = o + pv(1, vn1[...])
        o = o + pv(2, vp1[...])
        f[5].wait_recv()
        f[7].wait_recv()
        o = o + pv(3, vd[...])
        for g in range(G):
            b, hh = divmod(g, H)
            acc[b * SQ:(b + 1) * SQ, hh * DH:(hh + 1) * DH] = o[g]

        out_ref[...] = jnp.dot(acc[...], wo_ref[...],
                               preferred_element_type=jnp.float32)

        for i in range(8):
            f[i].wait_send()

    out2d = pl.pallas_call(
        body,
        out_shape=jax.ShapeDtypeStruct((B * SQ, D), jnp.float32),
        in_specs=[pl.BlockSpec(memory_space=pltpu.VMEM)] * 5,
        out_specs=pl.BlockSpec(memory_space=pltpu.VMEM),
        scratch_shapes=[
            pltpu.VMEM((B * SQ, D), jnp.float32),
            pltpu.VMEM((G, SQ, DH), jnp.float32),
            pltpu.VMEM((G, SQ, N_DEV * SKV), jnp.float32),
            pltpu.VMEM((B * SQ, D), jnp.float32),
            pltpu.VMEM((G * DH, SKV), jnp.float32),
            pltpu.VMEM((G * DH, SKV), jnp.float32),
            pltpu.VMEM((G * DH, SKV), jnp.float32),
            pltpu.VMEM((G * DH, SKV), jnp.float32),
            pltpu.VMEM((G * DH, SKV), jnp.float32),
            pltpu.VMEM((G * DH, SKV), jnp.float32),
            pltpu.SemaphoreType.DMA((8,)),
            pltpu.SemaphoreType.DMA((8,)),
        ],
        compiler_params=pltpu.CompilerParams(collective_id=0),
    )(x2d, Wq, Wo, kt, vt)
    return out2d.reshape(B, SQ, D)
